# baseline (speedup 1.0000x reference)
import functools

import jax
import jax.numpy as jnp
from jax import lax
from jax.experimental import pallas as pl
from jax.experimental.pallas import tpu as pltpu
from jax.experimental.pallas import tpu_sc as plsc

NC = 2
NS = 16
NW = NC * NS
CHUNK = 128
ZROWS = 128

_MESH = dict(core_axis_name="c", subcore_axis_name="s", num_cores=NC,
             num_subcores=NS)


CPP = 32
PH = 5


def _make_agg(n_pad, D, e_pad):
  T = e_pad // CHUNK
  unit = NS * CPP * PH
  assert T % unit == 0, (T, unit)
  ph0 = PH * (T // unit)
  n_pairs = CPP // 2
  rows_per_sub = n_pad // NS
  n_zc = rows_per_sub // ZROWS
  mesh = plsc.VectorSubcoreMesh(**_MESH)

  @functools.partial(
      pl.kernel,
      mesh=mesh,
      out_type=jax.ShapeDtypeStruct((n_pad, D), jnp.float32),
      scratch_types=[
          pltpu.VMEM((CPP * CHUNK,), jnp.int32),
          pltpu.VMEM((CPP, CHUNK), jnp.int32),
          pltpu.VMEM((CHUNK, D), jnp.float32),
          pltpu.VMEM((CHUNK, D), jnp.float32),
          pltpu.VMEM_SHARED((n_pad, D), jnp.float32),
          pltpu.SemaphoreType.DMA,
          pltpu.SemaphoreType.DMA,
      ],
  )
  def agg(p_hbm, src_hbm, dst2_hbm, out_hbm, src_v, dst_v, rows0, rows1,
          acc_sh, gsem0, gsem1):
    cid = lax.axis_index("c")
    sid = lax.axis_index("s")

    def gather_desc(c, rows, gsem):
      return pltpu.make_async_copy(
          p_hbm.at[src_v.at[pl.ds(c * CHUNK, CHUNK)]], rows, gsem)

    @pl.when(cid == 0)
    def _():
      zv = jnp.zeros((16,), jnp.float32)

      def zrow(i, carry):
        def zcol(c2, carry2):
          rows0[i, pl.ds(c2 * 16, 16)] = zv
          return carry2
        return lax.fori_loop(0, D // 16, zcol, carry)
      lax.fori_loop(0, CHUNK, zrow, 0)

      def zchunk(k2, carry):
        pltpu.sync_copy(rows0.at[pl.ds(0, ZROWS)],
                        acc_sh.at[pl.ds(sid * rows_per_sub + k2 * ZROWS,
                                        ZROWS)])
        return carry
      lax.fori_loop(0, n_zc, zchunk, 0)

      plsc.subcore_barrier()

      def phase(ph, carry):
        cbase = sid * (ph0 * CPP) + ph * CPP
        pltpu.sync_copy(src_hbm.at[pl.ds(cbase * CHUNK, CPP * CHUNK)], src_v)
        pltpu.sync_copy(dst2_hbm.at[pl.ds(cbase, CPP)], dst_v)
        gather_desc(0, rows0, gsem0).start()

        def pair(i, carry2):
          c0 = 2 * i
          gather_desc(c0 + 1, rows1, gsem1).start()
          gather_desc(c0, rows0, gsem0).wait()
          pltpu.sync_copy(rows0, acc_sh.at[dst_v.at[c0]], add=True)

          @pl.when(i + 1 < n_pairs)
          def _():
            gather_desc(c0 + 2, rows0, gsem0).start()
          gather_desc(c0 + 1, rows1, gsem1).wait()
          pltpu.sync_copy(rows1, acc_sh.at[dst_v.at[c0 + 1]], add=True)
          return carry2
        lax.fori_loop(0, n_pairs, pair, 0)
        return carry
      lax.fori_loop(0, ph0, phase, 0)

      plsc.subcore_barrier()
      pltpu.sync_copy(acc_sh.at[pl.ds(sid * rows_per_sub, rows_per_sub)],
                      out_hbm.at[pl.ds(sid * rows_per_sub, rows_per_sub)])

  return agg


def _make_deg(n_pad, e_pad):
  D = 128
  e_per_w = e_pad // NW
  n_chunks = e_per_w // CHUNK
  rows_per_sub = n_pad // NS
  n_zc = rows_per_sub // ZROWS
  mesh = plsc.VectorSubcoreMesh(**_MESH)

  @functools.partial(
      pl.kernel,
      mesh=mesh,
      out_type=jax.ShapeDtypeStruct((NC, n_pad, D), jnp.float32),
      scratch_types=[
          pltpu.VMEM((n_chunks, CHUNK), jnp.int32),
          pltpu.VMEM((CHUNK, D), jnp.float32),
          pltpu.VMEM((ZROWS, D), jnp.float32),
          pltpu.VMEM_SHARED((n_pad, D), jnp.float32),
      ],
  )
  def deg(dst2_hbm, out_hbm, dst_v, ones_v, zero_v, acc_sh):
    cid = lax.axis_index("c")
    sid = lax.axis_index("s")
    wid = sid * NC + cid

    onehot = jnp.where(lax.iota(jnp.int32, 16) == 0, 1.0, 0.0).astype(
        jnp.float32)
    zv = jnp.zeros((16,), jnp.float32)

    def frow(i, carry):
      def fcol(c2, carry2):
        ones_v[i, pl.ds(c2 * 16, 16)] = jnp.where(c2 == 0, onehot, zv)
        zero_v[i, pl.ds(c2 * 16, 16)] = zv
        return carry2
      return lax.fori_loop(0, D // 16, fcol, carry)
    lax.fori_loop(0, CHUNK, frow, 0)

    def zchunk(k, carry):
      pltpu.sync_copy(zero_v,
                      acc_sh.at[pl.ds(sid * rows_per_sub + k * ZROWS, ZROWS)])
      return carry
    lax.fori_loop(0, n_zc, zchunk, 0)

    pltpu.sync_copy(dst2_hbm.at[pl.ds(wid * n_chunks, n_chunks)], dst_v)

    plsc.subcore_barrier()

    def chunk(j, carry):
      pltpu.sync_copy(ones_v, acc_sh.at[dst_v.at[j]], add=True)
      return carry
    lax.fori_loop(0, n_chunks, chunk, 0)

    plsc.subcore_barrier()
    pltpu.sync_copy(acc_sh.at[pl.ds(sid * rows_per_sub, rows_per_sub)],
                    out_hbm.at[cid, pl.ds(sid * rows_per_sub, rows_per_sub)])

  return deg


def _dis_from_deg(deg_block):
  deg = jnp.sum(deg_block[0] + deg_block[1], axis=1) + 1.0
  return lax.rsqrt(deg)


def _prep(x_p, degpart):
  n_pad, F = x_p.shape
  B = 2048
  grid = n_pad // B

  def body(deg_ref, x_ref, px_ref):
    dis = _dis_from_deg(deg_ref[...])
    px_ref[...] = x_ref[...] * dis[:, None]

  return pl.pallas_call(
      body,
      grid=(grid,),
      in_specs=[
          pl.BlockSpec((2, B, 128), lambda i: (0, i, 0)),
          pl.BlockSpec((B, F), lambda i: (i, 0)),
      ],
      out_specs=pl.BlockSpec((B, F), lambda i: (i, 0)),
      out_shape=jax.ShapeDtypeStruct((n_pad, F), jnp.float32),
  )(degpart, x_p)


def _mlp(part, px, degpart, W1, b1, W2p):
  n_pad, F = px.shape
  H = W1.shape[1]
  D2 = W2p.shape[1]
  B = 1024
  grid = n_pad // B

  def body(part_ref, px_ref, deg_ref, w1_ref, b1_ref, w2_ref, out_ref):
    dis = _dis_from_deg(deg_ref[...])
    agg = (part_ref[...] + px_ref[...]) * dis[:, None]
    h1 = jnp.dot(agg, w1_ref[...], preferred_element_type=jnp.float32)
    h1 = jnp.maximum(h1 + b1_ref[...], 0.0)
    h2 = jnp.dot(h1, w2_ref[...], preferred_element_type=jnp.float32)
    out_ref[...] = h2 * dis[:, None]

  return pl.pallas_call(
      body,
      grid=(grid,),
      in_specs=[
          pl.BlockSpec((B, F), lambda i: (i, 0)),
          pl.BlockSpec((B, F), lambda i: (i, 0)),
          pl.BlockSpec((2, B, 128), lambda i: (0, i, 0)),
          pl.BlockSpec((F, H), lambda i: (0, 0)),
          pl.BlockSpec((1, H), lambda i: (0, 0)),
          pl.BlockSpec((H, D2), lambda i: (0, 0)),
      ],
      out_specs=pl.BlockSpec((B, D2), lambda i: (i, 0)),
      out_shape=jax.ShapeDtypeStruct((n_pad, D2), jnp.float32),
  )(part, px, degpart, W1, b1, W2p)


def _final(part2, ph2, degpart, b2p, n_out):
  n_pad, D2 = ph2.shape
  B = 2048
  grid = n_pad // B

  def body(part_ref, ph2_ref, deg_ref, b2_ref, out_ref):
    dis = _dis_from_deg(deg_ref[...])
    y = (part_ref[...] + ph2_ref[...]) * dis[:, None] + b2_ref[...]
    col = lax.broadcasted_iota(jnp.int32, (B, D2), 1)
    valid = col < n_out
    yv = jnp.where(valid, y, -1e30)
    m = jnp.max(yv, axis=1, keepdims=True)
    e = jnp.where(valid, jnp.exp(yv - m), 0.0)
    s = jnp.sum(e, axis=1, keepdims=True)
    out_ref[...] = y - m - jnp.log(s)

  return pl.pallas_call(
      body,
      grid=(grid,),
      in_specs=[
          pl.BlockSpec((B, D2), lambda i: (i, 0)),
          pl.BlockSpec((B, D2), lambda i: (i, 0)),
          pl.BlockSpec((2, B, 128), lambda i: (0, i, 0)),
          pl.BlockSpec((1, D2), lambda i: (0, 0)),
      ],
      out_specs=pl.BlockSpec((B, D2), lambda i: (i, 0)),
      out_shape=jax.ShapeDtypeStruct((n_pad, D2), jnp.float32),
  )(part2, ph2, degpart, b2p)


def _round_up(v, m):
  return (v + m - 1) // m * m


@jax.jit
def kernel(x, edge_index, W1, b1, W2, b2):
  n, in_f = x.shape
  e = edge_index.shape[1]
  n_pad = _round_up(n + 1, NS * ZROWS)
  e_pad = _round_up(e, NS * CPP * CHUNK * PH)
  d2 = _round_up(W2.shape[1], 128)

  src = edge_index[0].astype(jnp.int32)
  dst = edge_index[1].astype(jnp.int32)
  epad = jnp.full((e_pad - e,), n, jnp.int32)
  src_p = jnp.concatenate([src, epad])
  dst_p = jnp.concatenate([dst, epad])

  x_p = jnp.zeros((n_pad, in_f), jnp.float32).at[:n].set(x)
  W2p = jnp.zeros((W2.shape[0], d2), jnp.float32).at[:, :W2.shape[1]].set(W2)
  b1r = b1.reshape(1, -1)
  b2p = jnp.zeros((1, d2), jnp.float32).at[0, :W2.shape[1]].set(b2)

  dst2 = dst_p.reshape(e_pad // CHUNK, CHUNK)

  degpart = _make_deg(n_pad, e_pad)(dst2)
  px = _prep(x_p, degpart)
  part1 = _make_agg(n_pad, in_f, e_pad)(px, src_p, dst2)
  ph2 = _mlp(part1, px, degpart, W1, b1r, W2p)
  part2 = _make_agg(n_pad, d2, e_pad)(ph2, src_p, dst2)
  out = _final(part2, ph2, degpart, b2p, W2.shape[1])
  return out[:n, :W2.shape[1]]


# speedup vs baseline: 1.0025x; 1.0025x over previous
import functools

import jax
import jax.numpy as jnp
from jax import lax
from jax.experimental import pallas as pl
from jax.experimental.pallas import tpu as pltpu
from jax.experimental.pallas import tpu_sc as plsc

NC = 2
NS = 16
NW = NC * NS
CHUNK = 128
ZROWS = 128

_MESH = dict(core_axis_name="c", subcore_axis_name="s", num_cores=NC,
             num_subcores=NS)


CPP = 32
PH = 5


def _make_agg(n_pad, D, e_pad):
  T = e_pad // CHUNK
  unit = NS * CPP * PH
  assert T % unit == 0, (T, unit)
  ph0 = PH * (T // unit)
  n_pairs = CPP // 2
  rows_per_sub = n_pad // NS
  n_zc = rows_per_sub // ZROWS
  mesh = plsc.VectorSubcoreMesh(**_MESH)

  @functools.partial(
      pl.kernel,
      mesh=mesh,
      out_type=jax.ShapeDtypeStruct((n_pad, D), jnp.float32),
      scratch_types=[
          pltpu.VMEM((CPP * CHUNK,), jnp.int32),
          pltpu.VMEM((CPP, CHUNK), jnp.int32),
          pltpu.VMEM((CHUNK, D), jnp.float32),
          pltpu.VMEM((CHUNK, D), jnp.float32),
          pltpu.VMEM_SHARED((n_pad, D), jnp.float32),
          pltpu.SemaphoreType.DMA,
          pltpu.SemaphoreType.DMA,
      ],
  )
  def agg(p_hbm, src_hbm, dst2_hbm, out_hbm, src_v, dst_v, rows0, rows1,
          acc_sh, gsem0, gsem1):
    cid = lax.axis_index("c")
    sid = lax.axis_index("s")

    def gather_desc(c, rows, gsem):
      return pltpu.make_async_copy(
          p_hbm.at[src_v.at[pl.ds(c * CHUNK, CHUNK)]], rows, gsem)

    @pl.when(cid == 0)
    def _():
      zv = jnp.zeros((16,), jnp.float32)

      def zrow(i, carry):
        def zcol(c2, carry2):
          rows0[i, pl.ds(c2 * 16, 16)] = zv
          return carry2
        return lax.fori_loop(0, D // 16, zcol, carry)
      lax.fori_loop(0, CHUNK, zrow, 0)

      def zchunk(k2, carry):
        pltpu.sync_copy(rows0.at[pl.ds(0, ZROWS)],
                        acc_sh.at[pl.ds(sid * rows_per_sub + k2 * ZROWS,
                                        ZROWS)])
        return carry
      lax.fori_loop(0, n_zc, zchunk, 0)

      plsc.subcore_barrier()

      def phase(ph, carry):
        cbase = sid * (ph0 * CPP) + ph * CPP
        pltpu.sync_copy(src_hbm.at[pl.ds(cbase * CHUNK, CPP * CHUNK)], src_v)
        pltpu.sync_copy(dst2_hbm.at[pl.ds(cbase, CPP)], dst_v)
        gather_desc(0, rows0, gsem0).start()

        def pair(i, carry2):
          c0 = 2 * i
          gather_desc(c0 + 1, rows1, gsem1).start()
          gather_desc(c0, rows0, gsem0).wait()
          pltpu.sync_copy(rows0, acc_sh.at[dst_v.at[c0]], add=True)

          @pl.when(i + 1 < n_pairs)
          def _():
            gather_desc(c0 + 2, rows0, gsem0).start()
          gather_desc(c0 + 1, rows1, gsem1).wait()
          pltpu.sync_copy(rows1, acc_sh.at[dst_v.at[c0 + 1]], add=True)
          return carry2
        lax.fori_loop(0, n_pairs, pair, 0)
        return carry
      lax.fori_loop(0, ph0, phase, 0)

      plsc.subcore_barrier()
      pltpu.sync_copy(acc_sh.at[pl.ds(sid * rows_per_sub, rows_per_sub)],
                      out_hbm.at[pl.ds(sid * rows_per_sub, rows_per_sub)])

  return agg


def _make_deg(n_pad, e_pad):
  D = 128
  e_per_w = e_pad // NW
  n_chunks = e_per_w // CHUNK
  rows_per_sub = n_pad // NS
  n_zc = rows_per_sub // ZROWS
  mesh = plsc.VectorSubcoreMesh(**_MESH)

  @functools.partial(
      pl.kernel,
      mesh=mesh,
      out_type=jax.ShapeDtypeStruct((NC, n_pad, D), jnp.float32),
      scratch_types=[
          pltpu.VMEM((n_chunks, CHUNK), jnp.int32),
          pltpu.VMEM((CHUNK, D), jnp.float32),
          pltpu.VMEM((ZROWS, D), jnp.float32),
          pltpu.VMEM_SHARED((n_pad, D), jnp.float32),
      ],
  )
  def deg(dst2_hbm, out_hbm, dst_v, ones_v, zero_v, acc_sh):
    cid = lax.axis_index("c")
    sid = lax.axis_index("s")
    wid = sid * NC + cid

    onehot = jnp.where(lax.iota(jnp.int32, 16) == 0, 1.0, 0.0).astype(
        jnp.float32)
    zv = jnp.zeros((16,), jnp.float32)

    def frow(i, carry):
      def fcol(c2, carry2):
        ones_v[i, pl.ds(c2 * 16, 16)] = jnp.where(c2 == 0, onehot, zv)
        zero_v[i, pl.ds(c2 * 16, 16)] = zv
        return carry2
      return lax.fori_loop(0, D // 16, fcol, carry)
    lax.fori_loop(0, CHUNK, frow, 0)

    def zchunk(k, carry):
      pltpu.sync_copy(zero_v,
                      acc_sh.at[pl.ds(sid * rows_per_sub + k * ZROWS, ZROWS)])
      return carry
    lax.fori_loop(0, n_zc, zchunk, 0)

    pltpu.sync_copy(dst2_hbm.at[pl.ds(wid * n_chunks, n_chunks)], dst_v)

    plsc.subcore_barrier()

    def chunk(j, carry):
      pltpu.sync_copy(ones_v, acc_sh.at[dst_v.at[j]], add=True)
      return carry
    lax.fori_loop(0, n_chunks, chunk, 0)

    plsc.subcore_barrier()
    pltpu.sync_copy(acc_sh.at[pl.ds(sid * rows_per_sub, rows_per_sub)],
                    out_hbm.at[cid, pl.ds(sid * rows_per_sub, rows_per_sub)])

  return deg


def _dis_from_deg(deg_block):
  deg = jnp.sum(deg_block[0] + deg_block[1], axis=1) + 1.0
  return lax.rsqrt(deg)


def _prep(x_p, degpart):
  n_pad, F = x_p.shape
  B = 2048
  grid = n_pad // B

  def body(deg_ref, x_ref, px_ref):
    dis = _dis_from_deg(deg_ref[...])
    px_ref[...] = x_ref[...] * dis[:, None]

  return pl.pallas_call(
      body,
      grid=(grid,),
      in_specs=[
          pl.BlockSpec((2, B, 128), lambda i: (0, i, 0)),
          pl.BlockSpec((B, F), lambda i: (i, 0)),
      ],
      out_specs=pl.BlockSpec((B, F), lambda i: (i, 0)),
      out_shape=jax.ShapeDtypeStruct((n_pad, F), jnp.float32),
  )(degpart, x_p)


def _mlp(part, px, degpart, W1, b1, W2p):
  n_pad, F = px.shape
  H = W1.shape[1]
  D2 = W2p.shape[1]
  B = 1024
  grid = n_pad // B

  def body(part_ref, px_ref, deg_ref, w1_ref, b1_ref, w2_ref, out_ref):
    dis = _dis_from_deg(deg_ref[...])
    agg = (part_ref[...] + px_ref[...]) * dis[:, None]
    h1 = jnp.dot(agg, w1_ref[...], preferred_element_type=jnp.float32)
    h1 = jnp.maximum(h1 + b1_ref[...], 0.0)
    h2 = jnp.dot(h1, w2_ref[...], preferred_element_type=jnp.float32)
    out_ref[...] = h2 * dis[:, None]

  return pl.pallas_call(
      body,
      grid=(grid,),
      in_specs=[
          pl.BlockSpec((B, F), lambda i: (i, 0)),
          pl.BlockSpec((B, F), lambda i: (i, 0)),
          pl.BlockSpec((2, B, 128), lambda i: (0, i, 0)),
          pl.BlockSpec((F, H), lambda i: (0, 0)),
          pl.BlockSpec((1, H), lambda i: (0, 0)),
          pl.BlockSpec((H, D2), lambda i: (0, 0)),
      ],
      out_specs=pl.BlockSpec((B, D2), lambda i: (i, 0)),
      out_shape=jax.ShapeDtypeStruct((n_pad, D2), jnp.float32),
  )(part, px, degpart, W1, b1, W2p)


def _final(part2, ph2, degpart, b2p, n_out):
  n_pad, D2 = ph2.shape
  B = 2048
  grid = n_pad // B

  def body(part_ref, ph2_ref, deg_ref, b2_ref, out_ref):
    dis = _dis_from_deg(deg_ref[...])
    y = (part_ref[...] + ph2_ref[...]) * dis[:, None] + b2_ref[...]
    col = lax.broadcasted_iota(jnp.int32, (B, D2), 1)
    valid = col < n_out
    yv = jnp.where(valid, y, -1e30)
    m = jnp.max(yv, axis=1, keepdims=True)
    e = jnp.where(valid, jnp.exp(yv - m), 0.0)
    s = jnp.sum(e, axis=1, keepdims=True)
    out_ref[...] = y - m - jnp.log(s)

  return pl.pallas_call(
      body,
      grid=(grid,),
      in_specs=[
          pl.BlockSpec((B, D2), lambda i: (i, 0)),
          pl.BlockSpec((B, D2), lambda i: (i, 0)),
          pl.BlockSpec((2, B, 128), lambda i: (0, i, 0)),
          pl.BlockSpec((1, D2), lambda i: (0, 0)),
      ],
      out_specs=pl.BlockSpec((B, D2), lambda i: (i, 0)),
      out_shape=jax.ShapeDtypeStruct((n_pad, D2), jnp.float32),
  )(part2, ph2, degpart, b2p)


def _round_up(v, m):
  return (v + m - 1) // m * m


@jax.jit
def kernel(x, edge_index, W1, b1, W2, b2):
  n, in_f = x.shape
  e = edge_index.shape[1]
  n_pad = _round_up(n + 1, NS * ZROWS)
  e_pad = _round_up(e, NS * CPP * CHUNK * PH)
  d2 = _round_up(W2.shape[1], 128)

  src = edge_index[0].astype(jnp.int32)
  dst = edge_index[1].astype(jnp.int32)
  src_pad = jnp.full((e_pad - e,), n, jnp.int32)
  dst_pad = n + (jnp.arange(e_pad - e, dtype=jnp.int32) % (n_pad - n))
  src_p = jnp.concatenate([src, src_pad])
  dst_p = jnp.concatenate([dst, dst_pad])

  x_p = jnp.zeros((n_pad, in_f), jnp.float32).at[:n].set(x)
  W2p = jnp.zeros((W2.shape[0], d2), jnp.float32).at[:, :W2.shape[1]].set(W2)
  b1r = b1.reshape(1, -1)
  b2p = jnp.zeros((1, d2), jnp.float32).at[0, :W2.shape[1]].set(b2)

  dst2 = dst_p.reshape(e_pad // CHUNK, CHUNK)

  degpart = _make_deg(n_pad, e_pad)(dst2)
  px = _prep(x_p, degpart)
  part1 = _make_agg(n_pad, in_f, e_pad)(px, src_p, dst2)
  ph2 = _mlp(part1, px, degpart, W1, b1r, W2p)
  part2 = _make_agg(n_pad, d2, e_pad)(ph2, src_p, dst2)
  out = _final(part2, ph2, degpart, b2p, W2.shape[1])
  return out[:n, :W2.shape[1]]


# speedup vs baseline: 2.3990x; 2.3931x over previous
import functools

import jax
import jax.numpy as jnp
from jax import lax
from jax.experimental import pallas as pl
from jax.experimental.pallas import tpu as pltpu
from jax.experimental.pallas import tpu_sc as plsc

NC = 2
NS = 16
NW = NC * NS
CHUNK = 128
ZROWS = 128

_MESH = dict(core_axis_name="c", subcore_axis_name="s", num_cores=NC,
             num_subcores=NS)


CPP = 32
PH = 5


def _make_agg(n_pad, D, e_pad):
  T = e_pad // CHUNK
  unit = NS * CPP * PH
  assert T % unit == 0, (T, unit)
  ph0 = PH * (T // unit)
  n_pairs = CPP // 2
  rows_per_sub = n_pad // NS
  n_zc = rows_per_sub // ZROWS
  mesh = plsc.VectorSubcoreMesh(**_MESH)

  @functools.partial(
      pl.kernel,
      mesh=mesh,
      out_type=jax.ShapeDtypeStruct((n_pad, D), jnp.float32),
      scratch_types=[
          pltpu.VMEM((CPP * CHUNK,), jnp.int32),
          pltpu.VMEM((CPP, CHUNK), jnp.int32),
          pltpu.VMEM((CHUNK, D), jnp.float32),
          pltpu.VMEM((CHUNK, D), jnp.float32),
          pltpu.VMEM_SHARED((n_pad, D), jnp.float32),
          pltpu.SemaphoreType.DMA,
          pltpu.SemaphoreType.DMA,
      ],
  )
  def agg(p_hbm, src_hbm, dst2_hbm, out_hbm, src_v, dst_v, rows0, rows1,
          acc_sh, gsem0, gsem1):
    cid = lax.axis_index("c")
    sid = lax.axis_index("s")

    def gather_desc(c, rows, gsem):
      return pltpu.make_async_copy(
          p_hbm.at[src_v.at[pl.ds(c * CHUNK, CHUNK)]], rows, gsem)

    @pl.when(cid == 0)
    def _():
      zv = jnp.zeros((16,), jnp.float32)

      def zrow(i, carry):
        def zcol(c2, carry2):
          rows0[i, pl.ds(c2 * 16, 16)] = zv
          return carry2
        return lax.fori_loop(0, D // 16, zcol, carry)
      lax.fori_loop(0, CHUNK, zrow, 0)

      def zchunk(k2, carry):
        pltpu.sync_copy(rows0.at[pl.ds(0, ZROWS)],
                        acc_sh.at[pl.ds(sid * rows_per_sub + k2 * ZROWS,
                                        ZROWS)])
        return carry
      lax.fori_loop(0, n_zc, zchunk, 0)

      plsc.subcore_barrier()

      def phase(ph, carry):
        cbase = sid * (ph0 * CPP) + ph * CPP
        pltpu.sync_copy(src_hbm.at[pl.ds(cbase * CHUNK, CPP * CHUNK)], src_v)
        pltpu.sync_copy(dst2_hbm.at[pl.ds(cbase, CPP)], dst_v)
        gather_desc(0, rows0, gsem0).start()

        def pair(i, carry2):
          c0 = 2 * i
          gather_desc(c0 + 1, rows1, gsem1).start()
          gather_desc(c0, rows0, gsem0).wait()
          pltpu.sync_copy(rows0, acc_sh.at[dst_v.at[c0]], add=True)

          @pl.when(i + 1 < n_pairs)
          def _():
            gather_desc(c0 + 2, rows0, gsem0).start()
          gather_desc(c0 + 1, rows1, gsem1).wait()
          pltpu.sync_copy(rows1, acc_sh.at[dst_v.at[c0 + 1]], add=True)
          return carry2
        lax.fori_loop(0, n_pairs, pair, 0)
        return carry
      lax.fori_loop(0, ph0, phase, 0)

      plsc.subcore_barrier()
      pltpu.sync_copy(acc_sh.at[pl.ds(sid * rows_per_sub, rows_per_sub)],
                      out_hbm.at[pl.ds(sid * rows_per_sub, rows_per_sub)])

  return agg


def _make_deg(n_pad, e_pad):
  D = 128
  e_per_w = e_pad // NW
  n_chunks = e_per_w // CHUNK
  rows_per_sub = n_pad // NS
  n_zc = rows_per_sub // ZROWS
  mesh = plsc.VectorSubcoreMesh(**_MESH)

  @functools.partial(
      pl.kernel,
      mesh=mesh,
      out_type=jax.ShapeDtypeStruct((NC, n_pad, D), jnp.float32),
      scratch_types=[
          pltpu.VMEM((n_chunks, CHUNK), jnp.int32),
          pltpu.VMEM((CHUNK, D), jnp.float32),
          pltpu.VMEM((ZROWS, D), jnp.float32),
          pltpu.VMEM_SHARED((n_pad, D), jnp.float32),
      ],
  )
  def deg(dst2_hbm, out_hbm, dst_v, ones_v, zero_v, acc_sh):
    cid = lax.axis_index("c")
    sid = lax.axis_index("s")
    wid = sid * NC + cid

    onehot = jnp.where(lax.iota(jnp.int32, 16) == 0, 1.0, 0.0).astype(
        jnp.float32)
    zv = jnp.zeros((16,), jnp.float32)

    def frow(i, carry):
      def fcol(c2, carry2):
        ones_v[i, pl.ds(c2 * 16, 16)] = jnp.where(c2 == 0, onehot, zv)
        zero_v[i, pl.ds(c2 * 16, 16)] = zv
        return carry2
      return lax.fori_loop(0, D // 16, fcol, carry)
    lax.fori_loop(0, CHUNK, frow, 0)

    def zchunk(k, carry):
      pltpu.sync_copy(zero_v,
                      acc_sh.at[pl.ds(sid * rows_per_sub + k * ZROWS, ZROWS)])
      return carry
    lax.fori_loop(0, n_zc, zchunk, 0)

    pltpu.sync_copy(dst2_hbm.at[pl.ds(wid * n_chunks, n_chunks)], dst_v)

    plsc.subcore_barrier()

    def chunk(j, carry):
      pltpu.sync_copy(ones_v, acc_sh.at[dst_v.at[j]], add=True)
      return carry
    lax.fori_loop(0, n_chunks, chunk, 0)

    plsc.subcore_barrier()
    pltpu.sync_copy(acc_sh.at[pl.ds(sid * rows_per_sub, rows_per_sub)],
                    out_hbm.at[cid, pl.ds(sid * rows_per_sub, rows_per_sub)])

  return deg


def _dis_from_deg(deg_block):
  deg = jnp.sum(deg_block[0] + deg_block[1], axis=1) + 1.0
  return lax.rsqrt(deg)


def _prep(x_p, degpart):
  n_pad, F = x_p.shape
  B = 2048
  grid = n_pad // B

  def body(deg_ref, x_ref, px_ref):
    dis = _dis_from_deg(deg_ref[...])
    px_ref[...] = x_ref[...] * dis[:, None]

  return pl.pallas_call(
      body,
      grid=(grid,),
      in_specs=[
          pl.BlockSpec((2, B, 128), lambda i: (0, i, 0)),
          pl.BlockSpec((B, F), lambda i: (i, 0)),
      ],
      out_specs=pl.BlockSpec((B, F), lambda i: (i, 0)),
      out_shape=jax.ShapeDtypeStruct((n_pad, F), jnp.float32),
  )(degpart, x_p)


def _mlp(part, px, degpart, W1, b1, W2p):
  n_pad, F = px.shape
  H = W1.shape[1]
  D2 = W2p.shape[1]
  B = 1024
  grid = n_pad // B

  def body(part_ref, px_ref, deg_ref, w1_ref, b1_ref, w2_ref, out_ref):
    dis = _dis_from_deg(deg_ref[...])
    agg = (part_ref[...] + px_ref[...]) * dis[:, None]
    h1 = jnp.dot(agg, w1_ref[...], preferred_element_type=jnp.float32)
    h1 = jnp.maximum(h1 + b1_ref[...], 0.0)
    h2 = jnp.dot(h1, w2_ref[...], preferred_element_type=jnp.float32)
    out_ref[...] = h2 * dis[:, None]

  return pl.pallas_call(
      body,
      grid=(grid,),
      in_specs=[
          pl.BlockSpec((B, F), lambda i: (i, 0)),
          pl.BlockSpec((B, F), lambda i: (i, 0)),
          pl.BlockSpec((2, B, 128), lambda i: (0, i, 0)),
          pl.BlockSpec((F, H), lambda i: (0, 0)),
          pl.BlockSpec((1, H), lambda i: (0, 0)),
          pl.BlockSpec((H, D2), lambda i: (0, 0)),
      ],
      out_specs=pl.BlockSpec((B, D2), lambda i: (i, 0)),
      out_shape=jax.ShapeDtypeStruct((n_pad, D2), jnp.float32),
  )(part, px, degpart, W1, b1, W2p)


def _final(part2, ph2, degpart, b2p, n_out):
  n_pad, D2 = ph2.shape
  B = 2048
  grid = n_pad // B

  def body(part_ref, ph2_ref, deg_ref, b2_ref, out_ref):
    dis = _dis_from_deg(deg_ref[...])
    y = (part_ref[...] + ph2_ref[...]) * dis[:, None] + b2_ref[...]
    col = lax.broadcasted_iota(jnp.int32, (B, D2), 1)
    valid = col < n_out
    yv = jnp.where(valid, y, -1e30)
    m = jnp.max(yv, axis=1, keepdims=True)
    e = jnp.where(valid, jnp.exp(yv - m), 0.0)
    s = jnp.sum(e, axis=1, keepdims=True)
    out_ref[...] = y - m - jnp.log(s)

  return pl.pallas_call(
      body,
      grid=(grid,),
      in_specs=[
          pl.BlockSpec((B, D2), lambda i: (i, 0)),
          pl.BlockSpec((B, D2), lambda i: (i, 0)),
          pl.BlockSpec((2, B, 128), lambda i: (0, i, 0)),
          pl.BlockSpec((1, D2), lambda i: (0, 0)),
      ],
      out_specs=pl.BlockSpec((B, D2), lambda i: (i, 0)),
      out_shape=jax.ShapeDtypeStruct((n_pad, D2), jnp.float32),
  )(part2, ph2, degpart, b2p)


def _round_up(v, m):
  return (v + m - 1) // m * m


@jax.jit
def kernel(x, edge_index, W1, b1, W2, b2):
  n, in_f = x.shape
  e = edge_index.shape[1]
  n_pad = _round_up(n + 1, NS * ZROWS)
  e_pad = _round_up(e, NS * CPP * CHUNK * PH)
  d2 = _round_up(W2.shape[1], 128)

  src = edge_index[0].astype(jnp.int32)
  dst = edge_index[1].astype(jnp.int32)
  arange_pad = jnp.arange(e_pad - e, dtype=jnp.int32)
  src_pad = arange_pad % n
  dst_pad = n + (arange_pad % (n_pad - n))
  src_p = jnp.concatenate([src, src_pad])
  dst_p = jnp.concatenate([dst, dst_pad])

  x_p = jnp.zeros((n_pad, in_f), jnp.float32).at[:n].set(x)
  W2p = jnp.zeros((W2.shape[0], d2), jnp.float32).at[:, :W2.shape[1]].set(W2)
  b1r = b1.reshape(1, -1)
  b2p = jnp.zeros((1, d2), jnp.float32).at[0, :W2.shape[1]].set(b2)

  dst2 = dst_p.reshape(e_pad // CHUNK, CHUNK)

  degpart = _make_deg(n_pad, e_pad)(dst2)
  px = _prep(x_p, degpart)
  part1 = _make_agg(n_pad, in_f, e_pad)(px, src_p, dst2)
  ph2 = _mlp(part1, px, degpart, W1, b1r, W2p)
  part2 = _make_agg(n_pad, d2, e_pad)(ph2, src_p, dst2)
  out = _final(part2, ph2, degpart, b2p, W2.shape[1])
  return out[:n, :W2.shape[1]]


# speedup vs baseline: 3.6539x; 1.5231x over previous
import functools

import jax
import jax.numpy as jnp
from jax import lax
from jax.experimental import pallas as pl
from jax.experimental.pallas import tpu as pltpu
from jax.experimental.pallas import tpu_sc as plsc

NC = 2
NS = 16
NW = NC * NS
CHUNK = 128
ZROWS = 128

_MESH = dict(core_axis_name="c", subcore_axis_name="s", num_cores=NC,
             num_subcores=NS)


CPP = 40


def _make_agg(n_pad, D, e_pad):
  T = e_pad // CHUNK
  cps = T // NW
  assert cps % CPP == 0, (T, cps)
  n_ph = cps // CPP
  n_pairs = CPP // 2
  rows_per_sub = n_pad // NS
  n_zc = rows_per_sub // ZROWS
  mesh = plsc.VectorSubcoreMesh(**_MESH)

  @functools.partial(
      pl.kernel,
      mesh=mesh,
      out_type=jax.ShapeDtypeStruct((NC, n_pad, D), jnp.float32),
      scratch_types=[
          pltpu.VMEM((CPP * CHUNK,), jnp.int32),
          pltpu.VMEM((CPP, CHUNK), jnp.int32),
          pltpu.VMEM((CHUNK, D), jnp.float32),
          pltpu.VMEM((CHUNK, D), jnp.float32),
          pltpu.VMEM_SHARED((n_pad, D), jnp.float32),
          pltpu.SemaphoreType.DMA,
          pltpu.SemaphoreType.DMA,
      ],
  )
  def agg(p_hbm, src_hbm, dst2_hbm, out_hbm, src_v, dst_v, rows0, rows1,
          acc_sh, gsem0, gsem1):
    cid = lax.axis_index("c")
    sid = lax.axis_index("s")
    wid = cid * NS + sid

    def gather_desc(c, rows, gsem):
      return pltpu.make_async_copy(
          p_hbm.at[src_v.at[pl.ds(c * CHUNK, CHUNK)]], rows, gsem)

    zv = jnp.zeros((16,), jnp.float32)

    def zrow(i, carry):
      def zcol(c2, carry2):
        rows0[i, pl.ds(c2 * 16, 16)] = zv
        return carry2
      return lax.fori_loop(0, D // 16, zcol, carry)
    lax.fori_loop(0, CHUNK, zrow, 0)

    def zchunk(k2, carry):
      pltpu.sync_copy(rows0.at[pl.ds(0, ZROWS)],
                      acc_sh.at[pl.ds(sid * rows_per_sub + k2 * ZROWS,
                                      ZROWS)])
      return carry
    lax.fori_loop(0, n_zc, zchunk, 0)

    plsc.subcore_barrier()

    def phase(ph, carry):
      cbase = wid * cps + ph * CPP
      pltpu.sync_copy(src_hbm.at[pl.ds(cbase * CHUNK, CPP * CHUNK)], src_v)
      pltpu.sync_copy(dst2_hbm.at[pl.ds(cbase, CPP)], dst_v)
      gather_desc(0, rows0, gsem0).start()

      def pair(i, carry2):
        c0 = 2 * i
        gather_desc(c0 + 1, rows1, gsem1).start()
        gather_desc(c0, rows0, gsem0).wait()
        pltpu.sync_copy(rows0, acc_sh.at[dst_v.at[c0]], add=True)

        @pl.when(i + 1 < n_pairs)
        def _():
          gather_desc(c0 + 2, rows0, gsem0).start()
        gather_desc(c0 + 1, rows1, gsem1).wait()
        pltpu.sync_copy(rows1, acc_sh.at[dst_v.at[c0 + 1]], add=True)
        return carry2
      lax.fori_loop(0, n_pairs, pair, 0)
      return carry
    lax.fori_loop(0, n_ph, phase, 0)

    plsc.subcore_barrier()
    pltpu.sync_copy(acc_sh.at[pl.ds(sid * rows_per_sub, rows_per_sub)],
                    out_hbm.at[cid, pl.ds(sid * rows_per_sub, rows_per_sub)])

  return agg


def _make_deg(n_pad, e_pad):
  D = 128
  e_per_w = e_pad // NW
  n_chunks = e_per_w // CHUNK
  rows_per_sub = n_pad // NS
  n_zc = rows_per_sub // ZROWS
  mesh = plsc.VectorSubcoreMesh(**_MESH)

  @functools.partial(
      pl.kernel,
      mesh=mesh,
      out_type=jax.ShapeDtypeStruct((NC, n_pad, D), jnp.float32),
      scratch_types=[
          pltpu.VMEM((n_chunks, CHUNK), jnp.int32),
          pltpu.VMEM((CHUNK, D), jnp.float32),
          pltpu.VMEM((ZROWS, D), jnp.float32),
          pltpu.VMEM_SHARED((n_pad, D), jnp.float32),
      ],
  )
  def deg(dst2_hbm, out_hbm, dst_v, ones_v, zero_v, acc_sh):
    cid = lax.axis_index("c")
    sid = lax.axis_index("s")
    wid = sid * NC + cid

    onehot = jnp.where(lax.iota(jnp.int32, 16) == 0, 1.0, 0.0).astype(
        jnp.float32)
    zv = jnp.zeros((16,), jnp.float32)

    def frow(i, carry):
      def fcol(c2, carry2):
        ones_v[i, pl.ds(c2 * 16, 16)] = jnp.where(c2 == 0, onehot, zv)
        zero_v[i, pl.ds(c2 * 16, 16)] = zv
        return carry2
      return lax.fori_loop(0, D // 16, fcol, carry)
    lax.fori_loop(0, CHUNK, frow, 0)

    def zchunk(k, carry):
      pltpu.sync_copy(zero_v,
                      acc_sh.at[pl.ds(sid * rows_per_sub + k * ZROWS, ZROWS)])
      return carry
    lax.fori_loop(0, n_zc, zchunk, 0)

    pltpu.sync_copy(dst2_hbm.at[pl.ds(wid * n_chunks, n_chunks)], dst_v)

    plsc.subcore_barrier()

    def chunk(j, carry):
      pltpu.sync_copy(ones_v, acc_sh.at[dst_v.at[j]], add=True)
      return carry
    lax.fori_loop(0, n_chunks, chunk, 0)

    plsc.subcore_barrier()
    pltpu.sync_copy(acc_sh.at[pl.ds(sid * rows_per_sub, rows_per_sub)],
                    out_hbm.at[cid, pl.ds(sid * rows_per_sub, rows_per_sub)])

  return deg


def _dis_from_deg(deg_block):
  deg = jnp.sum(deg_block[0] + deg_block[1], axis=1) + 1.0
  return lax.rsqrt(deg)


def _prep(x_p, degpart):
  n_pad, F = x_p.shape
  B = 2048
  grid = n_pad // B

  def body(deg_ref, x_ref, px_ref):
    dis = _dis_from_deg(deg_ref[...])
    px_ref[...] = x_ref[...] * dis[:, None]

  return pl.pallas_call(
      body,
      grid=(grid,),
      in_specs=[
          pl.BlockSpec((2, B, 128), lambda i: (0, i, 0)),
          pl.BlockSpec((B, F), lambda i: (i, 0)),
      ],
      out_specs=pl.BlockSpec((B, F), lambda i: (i, 0)),
      out_shape=jax.ShapeDtypeStruct((n_pad, F), jnp.float32),
  )(degpart, x_p)


def _mlp(part, px, degpart, W1, b1, W2p):
  n_pad, F = px.shape
  H = W1.shape[1]
  D2 = W2p.shape[1]
  B = 1024
  grid = n_pad // B

  def body(part_ref, px_ref, deg_ref, w1_ref, b1_ref, w2_ref, out_ref):
    dis = _dis_from_deg(deg_ref[...])
    pr = part_ref[...]
    agg = (pr[0] + pr[1] + px_ref[...]) * dis[:, None]
    h1 = jnp.dot(agg, w1_ref[...], preferred_element_type=jnp.float32)
    h1 = jnp.maximum(h1 + b1_ref[...], 0.0)
    h2 = jnp.dot(h1, w2_ref[...], preferred_element_type=jnp.float32)
    out_ref[...] = h2 * dis[:, None]

  return pl.pallas_call(
      body,
      grid=(grid,),
      in_specs=[
          pl.BlockSpec((2, B, F), lambda i: (0, i, 0)),
          pl.BlockSpec((B, F), lambda i: (i, 0)),
          pl.BlockSpec((2, B, 128), lambda i: (0, i, 0)),
          pl.BlockSpec((F, H), lambda i: (0, 0)),
          pl.BlockSpec((1, H), lambda i: (0, 0)),
          pl.BlockSpec((H, D2), lambda i: (0, 0)),
      ],
      out_specs=pl.BlockSpec((B, D2), lambda i: (i, 0)),
      out_shape=jax.ShapeDtypeStruct((n_pad, D2), jnp.float32),
  )(part, px, degpart, W1, b1, W2p)


def _final(part2, ph2, degpart, b2p, n_out):
  n_pad, D2 = ph2.shape
  B = 2048
  grid = n_pad // B

  def body(part_ref, ph2_ref, deg_ref, b2_ref, out_ref):
    dis = _dis_from_deg(deg_ref[...])
    pr = part_ref[...]
    y = (pr[0] + pr[1] + ph2_ref[...]) * dis[:, None] + b2_ref[...]
    col = lax.broadcasted_iota(jnp.int32, (B, D2), 1)
    valid = col < n_out
    yv = jnp.where(valid, y, -1e30)
    m = jnp.max(yv, axis=1, keepdims=True)
    e = jnp.where(valid, jnp.exp(yv - m), 0.0)
    s = jnp.sum(e, axis=1, keepdims=True)
    out_ref[...] = y - m - jnp.log(s)

  return pl.pallas_call(
      body,
      grid=(grid,),
      in_specs=[
          pl.BlockSpec((2, B, D2), lambda i: (0, i, 0)),
          pl.BlockSpec((B, D2), lambda i: (i, 0)),
          pl.BlockSpec((2, B, 128), lambda i: (0, i, 0)),
          pl.BlockSpec((1, D2), lambda i: (0, 0)),
      ],
      out_specs=pl.BlockSpec((B, D2), lambda i: (i, 0)),
      out_shape=jax.ShapeDtypeStruct((n_pad, D2), jnp.float32),
  )(part2, ph2, degpart, b2p)


def _round_up(v, m):
  return (v + m - 1) // m * m


@jax.jit
def kernel(x, edge_index, W1, b1, W2, b2):
  n, in_f = x.shape
  e = edge_index.shape[1]
  n_pad = _round_up(n + 1, NS * ZROWS)
  e_pad = _round_up(e, NW * CPP * CHUNK)
  d2 = _round_up(W2.shape[1], 128)

  src = edge_index[0].astype(jnp.int32)
  dst = edge_index[1].astype(jnp.int32)
  arange_pad = jnp.arange(e_pad - e, dtype=jnp.int32)
  src_pad = arange_pad % n
  dst_pad = n + (arange_pad % (n_pad - n))
  src_p = jnp.concatenate([src, src_pad])
  dst_p = jnp.concatenate([dst, dst_pad])

  x_p = jnp.zeros((n_pad, in_f), jnp.float32).at[:n].set(x)
  W2p = jnp.zeros((W2.shape[0], d2), jnp.float32).at[:, :W2.shape[1]].set(W2)
  b1r = b1.reshape(1, -1)
  b2p = jnp.zeros((1, d2), jnp.float32).at[0, :W2.shape[1]].set(b2)

  dst2 = dst_p.reshape(e_pad // CHUNK, CHUNK)

  degpart = _make_deg(n_pad, e_pad)(dst2)
  px = _prep(x_p, degpart)
  part1 = _make_agg(n_pad, in_f, e_pad)(px, src_p, dst2)
  ph2 = _mlp(part1, px, degpart, W1, b1r, W2p)
  part2 = _make_agg(n_pad, d2, e_pad)(ph2, src_p, dst2)
  out = _final(part2, ph2, degpart, b2p, W2.shape[1])
  return out[:n, :W2.shape[1]]


# speedup vs baseline: 4.6136x; 1.2627x over previous
import functools

import jax
import jax.numpy as jnp
from jax import lax
from jax.experimental import pallas as pl
from jax.experimental.pallas import tpu as pltpu
from jax.experimental.pallas import tpu_sc as plsc

NC = 2
NS = 16
NW = NC * NS
CHUNK = 128
ZROWS = 128

_MESH = dict(core_axis_name="c", subcore_axis_name="s", num_cores=NC,
             num_subcores=NS)


CPP = 40


def _make_agg(n_pad, D, e_pad, tc_tiling=True):
  T = e_pad // CHUNK
  cps = T // NW
  assert cps % CPP == 0, (T, cps)
  n_ph = cps // CPP
  n_pairs = CPP // 2
  rows_per_sub = n_pad // NS
  n_zc = rows_per_sub // ZROWS
  mesh = plsc.VectorSubcoreMesh(**_MESH)

  @functools.partial(
      pl.kernel,
      mesh=mesh,
      out_type=jax.ShapeDtypeStruct((NC, n_pad, D), jnp.float32),
      compiler_params=pltpu.CompilerParams(use_tc_tiling_on_sc=tc_tiling),
      scratch_types=[
          pltpu.VMEM((CPP * CHUNK,), jnp.int32),
          pltpu.VMEM((CPP, CHUNK), jnp.int32),
          pltpu.VMEM((CHUNK, D), jnp.float32),
          pltpu.VMEM((CHUNK, D), jnp.float32),
          pltpu.VMEM_SHARED((n_pad, D), jnp.float32),
          pltpu.SemaphoreType.DMA,
          pltpu.SemaphoreType.DMA,
      ],
  )
  def agg(p_hbm, src_hbm, dst2_hbm, out_hbm, src_v, dst_v, rows0, rows1,
          acc_sh, gsem0, gsem1):
    cid = lax.axis_index("c")
    sid = lax.axis_index("s")
    wid = cid * NS + sid

    def gather_desc(c, rows, gsem):
      return pltpu.make_async_copy(
          p_hbm.at[src_v.at[pl.ds(c * CHUNK, CHUNK)]], rows, gsem)

    zv = jnp.zeros((16,), jnp.float32)

    def zrow(i, carry):
      def zcol(c2, carry2):
        rows0[i, pl.ds(c2 * 16, 16)] = zv
        return carry2
      return lax.fori_loop(0, D // 16, zcol, carry)
    lax.fori_loop(0, CHUNK, zrow, 0)

    def zchunk(k2, carry):
      pltpu.sync_copy(rows0.at[pl.ds(0, ZROWS)],
                      acc_sh.at[pl.ds(sid * rows_per_sub + k2 * ZROWS,
                                      ZROWS)])
      return carry
    lax.fori_loop(0, n_zc, zchunk, 0)

    plsc.subcore_barrier()

    def phase(ph, carry):
      cbase = wid * cps + ph * CPP
      pltpu.sync_copy(src_hbm.at[pl.ds(cbase * CHUNK, CPP * CHUNK)], src_v)
      pltpu.sync_copy(dst2_hbm.at[pl.ds(cbase, CPP)], dst_v)
      gather_desc(0, rows0, gsem0).start()

      def pair(i, carry2):
        c0 = 2 * i
        gather_desc(c0 + 1, rows1, gsem1).start()
        gather_desc(c0, rows0, gsem0).wait()
        pltpu.sync_copy(rows0, acc_sh.at[dst_v.at[c0]], add=True)

        @pl.when(i + 1 < n_pairs)
        def _():
          gather_desc(c0 + 2, rows0, gsem0).start()
        gather_desc(c0 + 1, rows1, gsem1).wait()
        pltpu.sync_copy(rows1, acc_sh.at[dst_v.at[c0 + 1]], add=True)
        return carry2
      lax.fori_loop(0, n_pairs, pair, 0)
      return carry
    lax.fori_loop(0, n_ph, phase, 0)

    plsc.subcore_barrier()
    pltpu.sync_copy(acc_sh.at[pl.ds(sid * rows_per_sub, rows_per_sub)],
                    out_hbm.at[cid, pl.ds(sid * rows_per_sub, rows_per_sub)])

  return agg


def _make_deg(n_pad, e_pad):
  D = 16
  e_per_w = e_pad // NW
  n_chunks = e_per_w // CHUNK
  rows_per_sub = n_pad // NS
  n_zc = rows_per_sub // ZROWS
  mesh = plsc.VectorSubcoreMesh(**_MESH)

  @functools.partial(
      pl.kernel,
      mesh=mesh,
      out_type=jax.ShapeDtypeStruct((NC, n_pad, D), jnp.float32),
      compiler_params=pltpu.CompilerParams(use_tc_tiling_on_sc=False),
      scratch_types=[
          pltpu.VMEM((n_chunks, CHUNK), jnp.int32),
          pltpu.VMEM((CHUNK, D), jnp.float32),
          pltpu.VMEM((ZROWS, D), jnp.float32),
          pltpu.VMEM_SHARED((n_pad, D), jnp.float32),
      ],
  )
  def deg(dst2_hbm, out_hbm, dst_v, ones_v, zero_v, acc_sh):
    cid = lax.axis_index("c")
    sid = lax.axis_index("s")
    wid = sid * NC + cid

    onehot = jnp.where(lax.iota(jnp.int32, 16) == 0, 1.0, 0.0).astype(
        jnp.float32)
    zv = jnp.zeros((16,), jnp.float32)

    def frow(i, carry):
      ones_v[i, pl.ds(0, 16)] = onehot
      zero_v[i, pl.ds(0, 16)] = zv
      return carry
    lax.fori_loop(0, CHUNK, frow, 0)

    def zchunk(k, carry):
      pltpu.sync_copy(zero_v,
                      acc_sh.at[pl.ds(sid * rows_per_sub + k * ZROWS, ZROWS)])
      return carry
    lax.fori_loop(0, n_zc, zchunk, 0)

    pltpu.sync_copy(dst2_hbm.at[pl.ds(wid * n_chunks, n_chunks)], dst_v)

    plsc.subcore_barrier()

    def chunk(j, carry):
      pltpu.sync_copy(ones_v, acc_sh.at[dst_v.at[j]], add=True)
      return carry
    lax.fori_loop(0, n_chunks, chunk, 0)

    plsc.subcore_barrier()
    pltpu.sync_copy(acc_sh.at[pl.ds(sid * rows_per_sub, rows_per_sub)],
                    out_hbm.at[cid, pl.ds(sid * rows_per_sub, rows_per_sub)])

  return deg


def _dis_from_deg(deg_block):
  deg = jnp.sum(deg_block[0] + deg_block[1], axis=1) + 1.0
  return lax.rsqrt(deg)


def _prep(x_p, degpart):
  n_pad, F = x_p.shape
  B = 2048
  grid = n_pad // B

  def body(deg_ref, x_ref, px_ref):
    dis = _dis_from_deg(deg_ref[...])
    px_ref[...] = x_ref[...] * dis[:, None]

  return pl.pallas_call(
      body,
      grid=(grid,),
      in_specs=[
          pl.BlockSpec((2, B, 16), lambda i: (0, i, 0)),
          pl.BlockSpec((B, F), lambda i: (i, 0)),
      ],
      out_specs=pl.BlockSpec((B, F), lambda i: (i, 0)),
      out_shape=jax.ShapeDtypeStruct((n_pad, F), jnp.float32),
  )(degpart, x_p)


def _mlp(part, px, degpart, W1, b1, W2p):
  n_pad, F = px.shape
  H = W1.shape[1]
  D2 = W2p.shape[1]
  B = 1024
  grid = n_pad // B

  def body(part_ref, px_ref, deg_ref, w1_ref, b1_ref, w2_ref, out_ref):
    dis = _dis_from_deg(deg_ref[...])
    pr = part_ref[...]
    agg = (pr[0] + pr[1] + px_ref[...]) * dis[:, None]
    h1 = jnp.dot(agg, w1_ref[...], preferred_element_type=jnp.float32)
    h1 = jnp.maximum(h1 + b1_ref[...], 0.0)
    h2 = jnp.dot(h1, w2_ref[...], preferred_element_type=jnp.float32)
    out_ref[...] = h2 * dis[:, None]

  return pl.pallas_call(
      body,
      grid=(grid,),
      in_specs=[
          pl.BlockSpec((2, B, F), lambda i: (0, i, 0)),
          pl.BlockSpec((B, F), lambda i: (i, 0)),
          pl.BlockSpec((2, B, 16), lambda i: (0, i, 0)),
          pl.BlockSpec((F, H), lambda i: (0, 0)),
          pl.BlockSpec((1, H), lambda i: (0, 0)),
          pl.BlockSpec((H, D2), lambda i: (0, 0)),
      ],
      out_specs=pl.BlockSpec((B, D2), lambda i: (i, 0)),
      out_shape=jax.ShapeDtypeStruct((n_pad, D2), jnp.float32),
  )(part, px, degpart, W1, b1, W2p)


def _final(part2, ph2, degpart, b2p, n_out):
  n_pad, D2 = ph2.shape
  B = 2048
  grid = n_pad // B

  def body(part_ref, ph2_ref, deg_ref, b2_ref, out_ref):
    dis = _dis_from_deg(deg_ref[...])
    pr = part_ref[...]
    y = (pr[0] + pr[1] + ph2_ref[...]) * dis[:, None] + b2_ref[...]
    col = lax.broadcasted_iota(jnp.int32, (B, D2), 1)
    valid = col < n_out
    yv = jnp.where(valid, y, -1e30)
    m = jnp.max(yv, axis=1, keepdims=True)
    e = jnp.where(valid, jnp.exp(yv - m), 0.0)
    s = jnp.sum(e, axis=1, keepdims=True)
    out_ref[...] = y - m - jnp.log(s)

  return pl.pallas_call(
      body,
      grid=(grid,),
      in_specs=[
          pl.BlockSpec((2, B, D2), lambda i: (0, i, 0)),
          pl.BlockSpec((B, D2), lambda i: (i, 0)),
          pl.BlockSpec((2, B, 16), lambda i: (0, i, 0)),
          pl.BlockSpec((1, D2), lambda i: (0, 0)),
      ],
      out_specs=pl.BlockSpec((B, D2), lambda i: (i, 0)),
      out_shape=jax.ShapeDtypeStruct((n_pad, D2), jnp.float32),
  )(part2, ph2, degpart, b2p)


def _round_up(v, m):
  return (v + m - 1) // m * m


@jax.jit
def kernel(x, edge_index, W1, b1, W2, b2):
  n, in_f = x.shape
  e = edge_index.shape[1]
  n_pad = _round_up(n + 1, NS * ZROWS)
  e_pad = _round_up(e, NW * CPP * CHUNK)
  d2 = _round_up(W2.shape[1], 16)

  src = edge_index[0].astype(jnp.int32)
  dst = edge_index[1].astype(jnp.int32)
  arange_pad = jnp.arange(e_pad - e, dtype=jnp.int32)
  src_pad = arange_pad % n
  dst_pad = n + (arange_pad % (n_pad - n))
  src_p = jnp.concatenate([src, src_pad])
  dst_p = jnp.concatenate([dst, dst_pad])

  x_p = jnp.zeros((n_pad, in_f), jnp.float32).at[:n].set(x)
  W2p = jnp.zeros((W2.shape[0], d2), jnp.float32).at[:, :W2.shape[1]].set(W2)
  b1r = b1.reshape(1, -1)
  b2p = jnp.zeros((1, d2), jnp.float32).at[0, :W2.shape[1]].set(b2)

  dst2 = dst_p.reshape(e_pad // CHUNK, CHUNK)

  degpart = _make_deg(n_pad, e_pad)(dst2)
  px = _prep(x_p, degpart)
  part1 = _make_agg(n_pad, in_f, e_pad)(px, src_p, dst2)
  ph2 = _mlp(part1, px, degpart, W1, b1r, W2p)
  part2 = _make_agg(n_pad, d2, e_pad, tc_tiling=False)(ph2, src_p, dst2)
  out = _final(part2, ph2, degpart, b2p, W2.shape[1])
  return out[:n, :W2.shape[1]]


# speedup vs baseline: 4.6527x; 1.0085x over previous
import functools

import jax
import jax.numpy as jnp
from jax import lax
from jax.experimental import pallas as pl
from jax.experimental.pallas import tpu as pltpu
from jax.experimental.pallas import tpu_sc as plsc

NC = 2
NS = 16
NW = NC * NS
CHUNK = 128
ZROWS = 128

_MESH = dict(core_axis_name="c", subcore_axis_name="s", num_cores=NC,
             num_subcores=NS)


CPP = 40


def _make_agg(n_pad, D, e_pad, tc_tiling=True):
  T = e_pad // CHUNK
  cps = T // NW
  assert cps % CPP == 0, (T, cps)
  n_ph = cps // CPP
  n_pairs = CPP // 2
  rows_per_sub = n_pad // NS
  n_zc = rows_per_sub // ZROWS
  mesh = plsc.VectorSubcoreMesh(**_MESH)

  @functools.partial(
      pl.kernel,
      mesh=mesh,
      out_type=jax.ShapeDtypeStruct((NC, n_pad, D), jnp.float32),
      compiler_params=pltpu.CompilerParams(use_tc_tiling_on_sc=tc_tiling),
      scratch_types=[
          pltpu.VMEM((CPP * CHUNK,), jnp.int32),
          pltpu.VMEM((CPP, CHUNK), jnp.int32),
          pltpu.VMEM((CHUNK, D), jnp.float32),
          pltpu.VMEM((CHUNK, D), jnp.float32),
          pltpu.VMEM_SHARED((n_pad, D), jnp.float32),
          pltpu.SemaphoreType.DMA,
          pltpu.SemaphoreType.DMA,
      ],
  )
  def agg(p_hbm, src_hbm, dst2_hbm, out_hbm, src_v, dst_v, rows0, rows1,
          acc_sh, gsem0, gsem1):
    cid = lax.axis_index("c")
    sid = lax.axis_index("s")
    wid = cid * NS + sid

    def gather_desc(c, rows, gsem):
      return pltpu.make_async_copy(
          p_hbm.at[src_v.at[pl.ds(c * CHUNK, CHUNK)]], rows, gsem)

    zv = jnp.zeros((16,), jnp.float32)

    def zrow(i, carry):
      def zcol(c2, carry2):
        rows0[i, pl.ds(c2 * 16, 16)] = zv
        return carry2
      return lax.fori_loop(0, D // 16, zcol, carry)
    lax.fori_loop(0, CHUNK, zrow, 0)

    def zchunk(k2, carry):
      pltpu.sync_copy(rows0.at[pl.ds(0, ZROWS)],
                      acc_sh.at[pl.ds(sid * rows_per_sub + k2 * ZROWS,
                                      ZROWS)])
      return carry
    lax.fori_loop(0, n_zc, zchunk, 0)

    plsc.subcore_barrier()

    def phase(ph, carry):
      cbase = wid * cps + ph * CPP
      pltpu.sync_copy(src_hbm.at[pl.ds(cbase * CHUNK, CPP * CHUNK)], src_v)
      pltpu.sync_copy(dst2_hbm.at[pl.ds(cbase, CPP)], dst_v)
      gather_desc(0, rows0, gsem0).start()

      def pair(i, carry2):
        c0 = 2 * i
        gather_desc(c0 + 1, rows1, gsem1).start()
        gather_desc(c0, rows0, gsem0).wait()
        pltpu.sync_copy(rows0, acc_sh.at[dst_v.at[c0]], add=True)

        @pl.when(i + 1 < n_pairs)
        def _():
          gather_desc(c0 + 2, rows0, gsem0).start()
        gather_desc(c0 + 1, rows1, gsem1).wait()
        pltpu.sync_copy(rows1, acc_sh.at[dst_v.at[c0 + 1]], add=True)
        return carry2
      lax.fori_loop(0, n_pairs, pair, 0)
      return carry
    lax.fori_loop(0, n_ph, phase, 0)

    plsc.subcore_barrier()
    pltpu.sync_copy(acc_sh.at[pl.ds(sid * rows_per_sub, rows_per_sub)],
                    out_hbm.at[cid, pl.ds(sid * rows_per_sub, rows_per_sub)])

  return agg


def _make_deg(n_pad, e_pad):
  D = 16
  e_per_w = e_pad // NW
  n_chunks = e_per_w // CHUNK
  rows_per_sub = n_pad // NS
  n_zc = rows_per_sub // ZROWS
  mesh = plsc.VectorSubcoreMesh(**_MESH)

  @functools.partial(
      pl.kernel,
      mesh=mesh,
      out_type=jax.ShapeDtypeStruct((NC, n_pad, D), jnp.float32),
      compiler_params=pltpu.CompilerParams(use_tc_tiling_on_sc=False),
      scratch_types=[
          pltpu.VMEM((n_chunks, CHUNK), jnp.int32),
          pltpu.VMEM((CHUNK, D), jnp.float32),
          pltpu.VMEM((ZROWS, D), jnp.float32),
          pltpu.VMEM_SHARED((n_pad, D), jnp.float32),
      ],
  )
  def deg(dst2_hbm, out_hbm, dst_v, ones_v, zero_v, acc_sh):
    cid = lax.axis_index("c")
    sid = lax.axis_index("s")
    wid = sid * NC + cid

    onehot = jnp.where(lax.iota(jnp.int32, 16) == 0, 1.0, 0.0).astype(
        jnp.float32)
    zv = jnp.zeros((16,), jnp.float32)

    def frow(i, carry):
      ones_v[i, pl.ds(0, 16)] = onehot
      zero_v[i, pl.ds(0, 16)] = zv
      return carry
    lax.fori_loop(0, CHUNK, frow, 0)

    def zchunk(k, carry):
      pltpu.sync_copy(zero_v,
                      acc_sh.at[pl.ds(sid * rows_per_sub + k * ZROWS, ZROWS)])
      return carry
    lax.fori_loop(0, n_zc, zchunk, 0)

    pltpu.sync_copy(dst2_hbm.at[pl.ds(wid * n_chunks, n_chunks)], dst_v)

    plsc.subcore_barrier()

    def chunk(j, carry):
      pltpu.sync_copy(ones_v, acc_sh.at[dst_v.at[j]], add=True)
      return carry
    lax.fori_loop(0, n_chunks, chunk, 0)

    plsc.subcore_barrier()
    pltpu.sync_copy(acc_sh.at[pl.ds(sid * rows_per_sub, rows_per_sub)],
                    out_hbm.at[cid, pl.ds(sid * rows_per_sub, rows_per_sub)])

  return deg


def _dis_from_deg(deg_block):
  deg = jnp.sum(deg_block[0] + deg_block[1], axis=1) + 1.0
  return lax.rsqrt(deg)


def _prep(x_p, degpart):
  n, F = x_p.shape
  B = 2000
  grid = n // B

  def body(deg_ref, x_ref, px_ref):
    dis = _dis_from_deg(deg_ref[...])
    px_ref[...] = x_ref[...] * dis[:, None]

  return pl.pallas_call(
      body,
      grid=(grid,),
      in_specs=[
          pl.BlockSpec((2, B, 16), lambda i: (0, i, 0)),
          pl.BlockSpec((B, F), lambda i: (i, 0)),
      ],
      out_specs=pl.BlockSpec((B, F), lambda i: (i, 0)),
      out_shape=jax.ShapeDtypeStruct((n, F), jnp.float32),
  )(degpart, x_p)


def _mlp(part, px, degpart, W1, b1, W2p):
  n, F = px.shape
  H = W1.shape[1]
  D2 = W2p.shape[1]
  B = 1000
  grid = n // B

  def body(part_ref, px_ref, deg_ref, w1_ref, b1_ref, w2_ref, out_ref):
    dis = _dis_from_deg(deg_ref[...])
    pr = part_ref[...]
    agg = (pr[0] + pr[1] + px_ref[...]) * dis[:, None]
    h1 = jnp.dot(agg, w1_ref[...], preferred_element_type=jnp.float32)
    h1 = jnp.maximum(h1 + b1_ref[...], 0.0)
    h2 = jnp.dot(h1, w2_ref[...], preferred_element_type=jnp.float32)
    out_ref[...] = h2 * dis[:, None]

  return pl.pallas_call(
      body,
      grid=(grid,),
      in_specs=[
          pl.BlockSpec((2, B, F), lambda i: (0, i, 0)),
          pl.BlockSpec((B, F), lambda i: (i, 0)),
          pl.BlockSpec((2, B, 16), lambda i: (0, i, 0)),
          pl.BlockSpec((F, H), lambda i: (0, 0)),
          pl.BlockSpec((1, H), lambda i: (0, 0)),
          pl.BlockSpec((H, D2), lambda i: (0, 0)),
      ],
      out_specs=pl.BlockSpec((B, D2), lambda i: (i, 0)),
      out_shape=jax.ShapeDtypeStruct((n, D2), jnp.float32),
  )(part, px, degpart, W1, b1, W2p)


def _final(part2, ph2, degpart, b2p, n_out):
  n, D2 = ph2.shape
  B = 2000
  grid = n // B

  def body(part_ref, ph2_ref, deg_ref, b2_ref, out_ref):
    dis = _dis_from_deg(deg_ref[...])
    pr = part_ref[...]
    y = (pr[0] + pr[1] + ph2_ref[...]) * dis[:, None] + b2_ref[...]
    col = lax.broadcasted_iota(jnp.int32, (B, D2), 1)
    valid = col < n_out
    yv = jnp.where(valid, y, -1e30)
    m = jnp.max(yv, axis=1, keepdims=True)
    e = jnp.where(valid, jnp.exp(yv - m), 0.0)
    s = jnp.sum(e, axis=1, keepdims=True)
    out_ref[...] = (y - m - jnp.log(s))[:, :n_out]

  return pl.pallas_call(
      body,
      grid=(grid,),
      in_specs=[
          pl.BlockSpec((2, B, D2), lambda i: (0, i, 0)),
          pl.BlockSpec((B, D2), lambda i: (i, 0)),
          pl.BlockSpec((2, B, 16), lambda i: (0, i, 0)),
          pl.BlockSpec((1, D2), lambda i: (0, 0)),
      ],
      out_specs=pl.BlockSpec((B, n_out), lambda i: (i, 0)),
      out_shape=jax.ShapeDtypeStruct((n, n_out), jnp.float32),
  )(part2, ph2, degpart, b2p)


def _round_up(v, m):
  return (v + m - 1) // m * m


@jax.jit
def kernel(x, edge_index, W1, b1, W2, b2):
  n, in_f = x.shape
  e = edge_index.shape[1]
  n_pad = _round_up(n + 1, NS * ZROWS)
  e_pad = _round_up(e, NW * CPP * CHUNK)
  d2 = _round_up(W2.shape[1], 16)

  src = edge_index[0].astype(jnp.int32)
  dst = edge_index[1].astype(jnp.int32)
  arange_pad = jnp.arange(e_pad - e, dtype=jnp.int32)
  src_pad = arange_pad % n
  dst_pad = n + (arange_pad % (n_pad - n))
  src_p = jnp.concatenate([src, src_pad])
  dst_p = jnp.concatenate([dst, dst_pad])

  W2p = jnp.zeros((W2.shape[0], d2), jnp.float32).at[:, :W2.shape[1]].set(W2)
  b1r = b1.reshape(1, -1)
  b2p = jnp.zeros((1, d2), jnp.float32).at[0, :W2.shape[1]].set(b2)

  dst2 = dst_p.reshape(e_pad // CHUNK, CHUNK)

  degpart = _make_deg(n_pad, e_pad)(dst2)
  px = _prep(x, degpart)
  part1 = _make_agg(n_pad, in_f, e_pad)(px, src_p, dst2)
  ph2 = _mlp(part1, px, degpart, W1, b1r, W2p)
  part2 = _make_agg(n_pad, d2, e_pad, tc_tiling=False)(ph2, src_p, dst2)
  return _final(part2, ph2, degpart, b2p, W2.shape[1])
